# baseline (speedup 1.0000x reference)
import numpy as np
import jax
import jax.numpy as jnp
from jax import lax
from jax.experimental import pallas as pl
from jax.experimental.pallas import tpu as pltpu
from jax.experimental.pallas import tpu_sc as plsc

_B = 128
_ROW = 3 * 224 * 224
_SUB = _ROW // 128


_COEFF_BITS = [
    0x3d8aa995, 0x3b0adbe7, 0x32a77334, 0x3ea2194a, 0x3ab17549, 0x3e4e2364, 0x3f702c15, 0x36a32374,
    0x3e1112ae, 0x3f536550, 0x3dad7116, 0x3f610a35, 0x359e53f7, 0x3f6da517, 0x3f79a195, 0x3f7d02d6,
    0x3953ac92, 0x3f37666c, 0x3d8122ac, 0x3a7c868c, 0x3a8c3175, 0x3e66a9a8, 0x3f743a3c, 0x3f7fbb4d,
    0x3f769b79, 0x3f7fadd5, 0x3f7ca80e, 0x3f7d50da, 0x3e9bf821, 0x3f7e24ad, 0x385c80f2, 0x3f22b615,
    0x3e5c4d0d, 0x3f7f857d, 0x33d443b5, 0x3eee7eb0, 0x3f7fffe3, 0x3d26cebd, 0x38c67df6, 0x3f800000,
    0x3f7ffc2b, 0x33023ed7, 0x3f707e58, 0x3bbea683, 0x3ba47304, 0x3f63b612, 0x3eba6d66, 0x3bdfc38c,
    0x3f38aa7a, 0x3c0c7b62, 0x3dca5593, 0x3ec0ff48, 0x3f7fe5c3, 0x3e06f165, 0x3f796886, 0x3f49b8bd,
    0x3c23c8ab, 0x3b0c9b72, 0x2c72ff2c, 0x3f1a2af9, 0x3e2f970f, 0x3dde831c, 0x3f338f86, 0x3c43619c,
    0x3e1bc035, 0x3e9d6340, 0x3dd131a1, 0x39ff8fbf, 0x3d0db273, 0x3befa030, 0x3929564a, 0x3d63e6a1,
    0x3b681477, 0x3f7b5d60, 0x3dc9b188, 0x3f5c5b75, 0x3f7d7200, 0x3f503da5, 0x3ea70a9b, 0x3c3028b7,
    0x3da1c41c, 0x3aaf8c13, 0x3668c158, 0x2edad15c, 0x3f7efe4f, 0x3f67742e, 0x3f7e7a0e, 0x3cee4e69,
    0x3f6930dc, 0x3dab8b25, 0x3e188c64, 0x3a49b09e, 0x3f7d6765, 0x3d83424c, 0x3f150e74, 0x32d9477f,
    0x3f7be310, 0x3f79ec19, 0x3f7ffff9, 0x3f7fc3ab, 0x3f759076, 0x39735b79, 0x38aebadf, 0x3d699950,
    0x3e9f28b5, 0x3c7f65ec, 0x3f6a68f7, 0x3f7f1e62, 0x3f478cd4, 0x3f5cb538, 0x378da790, 0x3587b406,
    0x3d9e3e03, 0x3b1212b5, 0x3f7327ff, 0x3e22a57b, 0x3b359439, 0x3f78820e, 0x3f674a2a, 0x3e940a6a,
    0x3d7dac54, 0x3746b599, 0x3f7dae59, 0x3d837b1d, 0x3f7da727, 0x3c67fac8, 0x3e8294fd, 0x3ec57bc7,
]
_COEFFS = np.array(_COEFF_BITS, dtype=np.uint32).view(np.float32)
_PERM = np.array([
    83, 2, 65, 73, 78, 32, 15, 10, 71, 48, 85, 25, 116, 109, 114, 115,
    77, 28, 106, 93, 92, 0, 82, 49, 69, 87, 89, 104, 75, 4, 90, 60,
    84, 42, 21, 112, 72, 11, 20, 74, 103, 57, 17, 12, 125, 19, 22, 67,
    97, 18, 16, 27, 5, 86, 99, 23, 39, 100, 111, 26, 122, 7, 102, 29,
    126, 117, 98, 70, 120, 54, 9, 88, 96, 41, 53, 81, 13, 124, 105, 80,
    36, 37, 34, 6, 95, 46, 108, 62, 3, 52, 14, 66, 1, 123, 76, 61,
    110, 40, 44, 8, 58, 47, 33, 38, 55, 31, 119, 101, 118, 68, 64, 91,
    51, 79, 63, 24, 56, 107, 43, 127, 30, 121, 59, 94, 45, 113, 35, 50,
], dtype=np.int32)


_CHUNK = 24


def _x_slab_body(x_ref, o_ref):
    for i in range(_B):
        c = float(_COEFFS[i])
        p = int(_PERM[i])
        o_ref[i] = c * x_ref[i] + (1.0 - c) * x_ref[p]


_YD = 1000
_NW = 32
_RPW = _B // _NW
_FL = _RPW * _YD

_IDX_TAB = np.tile(_PERM.reshape(_NW, _RPW)[:, :1], (1, 16)).astype(np.int32)
_IDX_TAB[:, :_RPW] = _PERM.reshape(_NW, _RPW)
_CC_FULL = np.repeat(_COEFFS.reshape(_B, 1), _YD, axis=1)

_YOFFS = [16 * j for j in range(_YD // 16)] + [_YD - 16]


def _y_sc_body(y_hbm, ypad_hbm, idx_hbm, cc_hbm, o_hbm,
               idx_v, a_v, b_v, cc_v, o_v, sem):
    w = lax.axis_index("s") * 2 + lax.axis_index("c")
    base = w * _RPW
    pltpu.sync_copy(idx_hbm.at[w], idx_v)
    pltpu.sync_copy(y_hbm.at[pl.ds(base, _RPW)], a_v)
    pltpu.sync_copy(cc_hbm.at[pl.ds(base, _RPW)], cc_v)
    pltpu.async_copy(ypad_hbm.at[idx_v], b_v, sem).wait()
    for r in range(_RPW):
        for off in _YOFFS:
            sl = pl.ds(off, 16)
            cc = cc_v[r, sl]
            o_v[r, sl] = cc * a_v[r, sl] + (1.0 - cc) * b_v[r, sl]
    pltpu.sync_copy(o_v, o_hbm.at[pl.ds(base, _RPW)])


def kernel(X, Y):
    X3 = X.reshape(_B, _SUB, 128)
    x_out = pl.pallas_call(
        _x_slab_body,
        grid=(_SUB // _CHUNK,),
        in_specs=[pl.BlockSpec((_B, _CHUNK, 128), lambda k: (0, k, 0))],
        out_specs=pl.BlockSpec((_B, _CHUNK, 128), lambda k: (0, k, 0)),
        out_shape=jax.ShapeDtypeStruct((_B, _SUB, 128), jnp.float32),
        compiler_params=pltpu.CompilerParams(
            dimension_semantics=("arbitrary",),
        ),
    )(X3)

    y_pad = jnp.pad(Y, ((0, 0), (0, 1024 - _YD)))
    y_out = pl.kernel(
        _y_sc_body,
        mesh=plsc.VectorSubcoreMesh(core_axis_name="c", subcore_axis_name="s"),
        out_type=jax.ShapeDtypeStruct((_B, _YD), jnp.float32),
        scratch_types=[
            pltpu.VMEM((16,), jnp.int32),
            pltpu.VMEM((_RPW, _YD), jnp.float32),
            pltpu.VMEM((16, 1024), jnp.float32),
            pltpu.VMEM((_RPW, _YD), jnp.float32),
            pltpu.VMEM((_RPW, _YD), jnp.float32),
            pltpu.SemaphoreType.DMA,
        ],
    )(Y, y_pad, jnp.asarray(_IDX_TAB), jnp.asarray(_CC_FULL))

    return (x_out.reshape(X.shape), y_out)


# speedup vs baseline: 1.0599x; 1.0599x over previous
import numpy as np
import jax
import jax.numpy as jnp
from jax import lax
from jax.experimental import pallas as pl
from jax.experimental.pallas import tpu as pltpu
from jax.experimental.pallas import tpu_sc as plsc

_B = 128
_ROW = 3 * 224 * 224
_SUB = _ROW // 128


_COEFF_BITS = [
    0x3d8aa995, 0x3b0adbe7, 0x32a77334, 0x3ea2194a, 0x3ab17549, 0x3e4e2364, 0x3f702c15, 0x36a32374,
    0x3e1112ae, 0x3f536550, 0x3dad7116, 0x3f610a35, 0x359e53f7, 0x3f6da517, 0x3f79a195, 0x3f7d02d6,
    0x3953ac92, 0x3f37666c, 0x3d8122ac, 0x3a7c868c, 0x3a8c3175, 0x3e66a9a8, 0x3f743a3c, 0x3f7fbb4d,
    0x3f769b79, 0x3f7fadd5, 0x3f7ca80e, 0x3f7d50da, 0x3e9bf821, 0x3f7e24ad, 0x385c80f2, 0x3f22b615,
    0x3e5c4d0d, 0x3f7f857d, 0x33d443b5, 0x3eee7eb0, 0x3f7fffe3, 0x3d26cebd, 0x38c67df6, 0x3f800000,
    0x3f7ffc2b, 0x33023ed7, 0x3f707e58, 0x3bbea683, 0x3ba47304, 0x3f63b612, 0x3eba6d66, 0x3bdfc38c,
    0x3f38aa7a, 0x3c0c7b62, 0x3dca5593, 0x3ec0ff48, 0x3f7fe5c3, 0x3e06f165, 0x3f796886, 0x3f49b8bd,
    0x3c23c8ab, 0x3b0c9b72, 0x2c72ff2c, 0x3f1a2af9, 0x3e2f970f, 0x3dde831c, 0x3f338f86, 0x3c43619c,
    0x3e1bc035, 0x3e9d6340, 0x3dd131a1, 0x39ff8fbf, 0x3d0db273, 0x3befa030, 0x3929564a, 0x3d63e6a1,
    0x3b681477, 0x3f7b5d60, 0x3dc9b188, 0x3f5c5b75, 0x3f7d7200, 0x3f503da5, 0x3ea70a9b, 0x3c3028b7,
    0x3da1c41c, 0x3aaf8c13, 0x3668c158, 0x2edad15c, 0x3f7efe4f, 0x3f67742e, 0x3f7e7a0e, 0x3cee4e69,
    0x3f6930dc, 0x3dab8b25, 0x3e188c64, 0x3a49b09e, 0x3f7d6765, 0x3d83424c, 0x3f150e74, 0x32d9477f,
    0x3f7be310, 0x3f79ec19, 0x3f7ffff9, 0x3f7fc3ab, 0x3f759076, 0x39735b79, 0x38aebadf, 0x3d699950,
    0x3e9f28b5, 0x3c7f65ec, 0x3f6a68f7, 0x3f7f1e62, 0x3f478cd4, 0x3f5cb538, 0x378da790, 0x3587b406,
    0x3d9e3e03, 0x3b1212b5, 0x3f7327ff, 0x3e22a57b, 0x3b359439, 0x3f78820e, 0x3f674a2a, 0x3e940a6a,
    0x3d7dac54, 0x3746b599, 0x3f7dae59, 0x3d837b1d, 0x3f7da727, 0x3c67fac8, 0x3e8294fd, 0x3ec57bc7,
]
_COEFFS = np.array(_COEFF_BITS, dtype=np.uint32).view(np.float32)
_PERM = np.array([
    83, 2, 65, 73, 78, 32, 15, 10, 71, 48, 85, 25, 116, 109, 114, 115,
    77, 28, 106, 93, 92, 0, 82, 49, 69, 87, 89, 104, 75, 4, 90, 60,
    84, 42, 21, 112, 72, 11, 20, 74, 103, 57, 17, 12, 125, 19, 22, 67,
    97, 18, 16, 27, 5, 86, 99, 23, 39, 100, 111, 26, 122, 7, 102, 29,
    126, 117, 98, 70, 120, 54, 9, 88, 96, 41, 53, 81, 13, 124, 105, 80,
    36, 37, 34, 6, 95, 46, 108, 62, 3, 52, 14, 66, 1, 123, 76, 61,
    110, 40, 44, 8, 58, 47, 33, 38, 55, 31, 119, 101, 118, 68, 64, 91,
    51, 79, 63, 24, 56, 107, 43, 127, 30, 121, 59, 94, 45, 113, 35, 50,
], dtype=np.int32)


_CHUNK = 56


def _x_slab_body(x_ref, o_ref):
    for i in range(_B):
        c = float(_COEFFS[i])
        p = int(_PERM[i])
        o_ref[i] = c * x_ref[i] + (1.0 - c) * x_ref[p]


_YD = 1000
_NW = 32
_RPW = _B // _NW
_FL = _RPW * _YD

_IDX_TAB = np.tile(_PERM.reshape(_NW, _RPW)[:, :1], (1, 16)).astype(np.int32)
_IDX_TAB[:, :_RPW] = _PERM.reshape(_NW, _RPW)
_CC_FULL = np.repeat(_COEFFS.reshape(_B, 1), _YD, axis=1)

_YOFFS = [16 * j for j in range(_YD // 16)] + [_YD - 16]


def _y_sc_body(y_hbm, ypad_hbm, idx_hbm, cc_hbm, o_hbm,
               idx_v, a_v, b_v, cc_v, o_v, sem):
    w = lax.axis_index("s") * 2 + lax.axis_index("c")
    base = w * _RPW
    pltpu.sync_copy(idx_hbm.at[w], idx_v)
    pltpu.sync_copy(y_hbm.at[pl.ds(base, _RPW)], a_v)
    pltpu.sync_copy(cc_hbm.at[pl.ds(base, _RPW)], cc_v)
    pltpu.async_copy(ypad_hbm.at[idx_v], b_v, sem).wait()
    for r in range(_RPW):
        for off in _YOFFS:
            sl = pl.ds(off, 16)
            cc = cc_v[r, sl]
            o_v[r, sl] = cc * a_v[r, sl] + (1.0 - cc) * b_v[r, sl]
    pltpu.sync_copy(o_v, o_hbm.at[pl.ds(base, _RPW)])


def kernel(X, Y):
    X3 = X.reshape(_B, _SUB, 128)
    x_out = pl.pallas_call(
        _x_slab_body,
        grid=(_SUB // _CHUNK,),
        in_specs=[pl.BlockSpec((_B, _CHUNK, 128), lambda k: (0, k, 0))],
        out_specs=pl.BlockSpec((_B, _CHUNK, 128), lambda k: (0, k, 0)),
        out_shape=jax.ShapeDtypeStruct((_B, _SUB, 128), jnp.float32),
        compiler_params=pltpu.CompilerParams(
            dimension_semantics=("parallel",),
        ),
    )(X3)

    y_pad = jnp.pad(Y, ((0, 0), (0, 1024 - _YD)))
    y_out = pl.kernel(
        _y_sc_body,
        mesh=plsc.VectorSubcoreMesh(core_axis_name="c", subcore_axis_name="s"),
        out_type=jax.ShapeDtypeStruct((_B, _YD), jnp.float32),
        cost_estimate=pl.CostEstimate(
            flops=3 * _B * _YD, transcendentals=0,
            bytes_accessed=4 * 4 * _B * _YD),
        scratch_types=[
            pltpu.VMEM((16,), jnp.int32),
            pltpu.VMEM((_RPW, _YD), jnp.float32),
            pltpu.VMEM((16, 1024), jnp.float32),
            pltpu.VMEM((_RPW, _YD), jnp.float32),
            pltpu.VMEM((_RPW, _YD), jnp.float32),
            pltpu.SemaphoreType.DMA,
        ],
    )(Y, y_pad, jnp.asarray(_IDX_TAB), jnp.asarray(_CC_FULL))

    return (x_out.reshape(X.shape), y_out)


# speedup vs baseline: 1.0635x; 1.0033x over previous
import numpy as np
import jax
import jax.numpy as jnp
from jax import lax
from jax.experimental import pallas as pl
from jax.experimental.pallas import tpu as pltpu
from jax.experimental.pallas import tpu_sc as plsc

_B = 128
_ROW = 3 * 224 * 224
_SUB = _ROW // 128


_COEFF_BITS = [
    0x3d8aa995, 0x3b0adbe7, 0x32a77334, 0x3ea2194a, 0x3ab17549, 0x3e4e2364, 0x3f702c15, 0x36a32374,
    0x3e1112ae, 0x3f536550, 0x3dad7116, 0x3f610a35, 0x359e53f7, 0x3f6da517, 0x3f79a195, 0x3f7d02d6,
    0x3953ac92, 0x3f37666c, 0x3d8122ac, 0x3a7c868c, 0x3a8c3175, 0x3e66a9a8, 0x3f743a3c, 0x3f7fbb4d,
    0x3f769b79, 0x3f7fadd5, 0x3f7ca80e, 0x3f7d50da, 0x3e9bf821, 0x3f7e24ad, 0x385c80f2, 0x3f22b615,
    0x3e5c4d0d, 0x3f7f857d, 0x33d443b5, 0x3eee7eb0, 0x3f7fffe3, 0x3d26cebd, 0x38c67df6, 0x3f800000,
    0x3f7ffc2b, 0x33023ed7, 0x3f707e58, 0x3bbea683, 0x3ba47304, 0x3f63b612, 0x3eba6d66, 0x3bdfc38c,
    0x3f38aa7a, 0x3c0c7b62, 0x3dca5593, 0x3ec0ff48, 0x3f7fe5c3, 0x3e06f165, 0x3f796886, 0x3f49b8bd,
    0x3c23c8ab, 0x3b0c9b72, 0x2c72ff2c, 0x3f1a2af9, 0x3e2f970f, 0x3dde831c, 0x3f338f86, 0x3c43619c,
    0x3e1bc035, 0x3e9d6340, 0x3dd131a1, 0x39ff8fbf, 0x3d0db273, 0x3befa030, 0x3929564a, 0x3d63e6a1,
    0x3b681477, 0x3f7b5d60, 0x3dc9b188, 0x3f5c5b75, 0x3f7d7200, 0x3f503da5, 0x3ea70a9b, 0x3c3028b7,
    0x3da1c41c, 0x3aaf8c13, 0x3668c158, 0x2edad15c, 0x3f7efe4f, 0x3f67742e, 0x3f7e7a0e, 0x3cee4e69,
    0x3f6930dc, 0x3dab8b25, 0x3e188c64, 0x3a49b09e, 0x3f7d6765, 0x3d83424c, 0x3f150e74, 0x32d9477f,
    0x3f7be310, 0x3f79ec19, 0x3f7ffff9, 0x3f7fc3ab, 0x3f759076, 0x39735b79, 0x38aebadf, 0x3d699950,
    0x3e9f28b5, 0x3c7f65ec, 0x3f6a68f7, 0x3f7f1e62, 0x3f478cd4, 0x3f5cb538, 0x378da790, 0x3587b406,
    0x3d9e3e03, 0x3b1212b5, 0x3f7327ff, 0x3e22a57b, 0x3b359439, 0x3f78820e, 0x3f674a2a, 0x3e940a6a,
    0x3d7dac54, 0x3746b599, 0x3f7dae59, 0x3d837b1d, 0x3f7da727, 0x3c67fac8, 0x3e8294fd, 0x3ec57bc7,
]
_COEFFS = np.array(_COEFF_BITS, dtype=np.uint32).view(np.float32)
_PERM = np.array([
    83, 2, 65, 73, 78, 32, 15, 10, 71, 48, 85, 25, 116, 109, 114, 115,
    77, 28, 106, 93, 92, 0, 82, 49, 69, 87, 89, 104, 75, 4, 90, 60,
    84, 42, 21, 112, 72, 11, 20, 74, 103, 57, 17, 12, 125, 19, 22, 67,
    97, 18, 16, 27, 5, 86, 99, 23, 39, 100, 111, 26, 122, 7, 102, 29,
    126, 117, 98, 70, 120, 54, 9, 88, 96, 41, 53, 81, 13, 124, 105, 80,
    36, 37, 34, 6, 95, 46, 108, 62, 3, 52, 14, 66, 1, 123, 76, 61,
    110, 40, 44, 8, 58, 47, 33, 38, 55, 31, 119, 101, 118, 68, 64, 91,
    51, 79, 63, 24, 56, 107, 43, 127, 30, 121, 59, 94, 45, 113, 35, 50,
], dtype=np.int32)


_CHUNK = 56


def _x_slab_body(x_ref, o_ref):
    for i in range(_B):
        c = float(_COEFFS[i])
        p = int(_PERM[i])
        o_ref[i] = c * x_ref[i] + (1.0 - c) * x_ref[p]


_YD = 1000
_NW = 32
_RPW = _B // _NW
_FL = _RPW * _YD

_IDX_TAB = np.tile(_PERM.reshape(_NW, _RPW)[:, :1], (1, 16)).astype(np.int32)
_IDX_TAB[:, :_RPW] = _PERM.reshape(_NW, _RPW)
_CC_FULL = np.repeat(_COEFFS.reshape(_B, 1), _YD, axis=1)

_YOFFS = [16 * j for j in range(_YD // 16)] + [_YD - 16]


def _y_sc_body(y_hbm, ypad_hbm, idx_hbm, cc_hbm, o_hbm,
               idx_v, a_v, b_v, cc_v, o_v, sem):
    w = lax.axis_index("s") * 2 + lax.axis_index("c")
    base = w * _RPW
    pltpu.sync_copy(idx_hbm.at[w], idx_v)
    pltpu.sync_copy(y_hbm.at[pl.ds(base, _RPW)], a_v)
    pltpu.sync_copy(cc_hbm.at[pl.ds(base, _RPW)], cc_v)
    pltpu.async_copy(ypad_hbm.at[idx_v], b_v, sem).wait()
    for r in range(_RPW):
        for off in _YOFFS:
            sl = pl.ds(off, 16)
            cc = cc_v[r, sl]
            o_v[r, sl] = cc * a_v[r, sl] + (1.0 - cc) * b_v[r, sl]
    pltpu.sync_copy(o_v, o_hbm.at[pl.ds(base, _RPW)])


def kernel(X, Y):
    y_pad = jnp.pad(Y, ((0, 0), (0, 1024 - _YD)))
    y_out = pl.kernel(
        _y_sc_body,
        mesh=plsc.VectorSubcoreMesh(core_axis_name="c", subcore_axis_name="s"),
        out_type=jax.ShapeDtypeStruct((_B, _YD), jnp.float32),
        cost_estimate=pl.CostEstimate(
            flops=3 * _B * _YD, transcendentals=0,
            bytes_accessed=4 * 4 * _B * _YD),
        scratch_types=[
            pltpu.VMEM((16,), jnp.int32),
            pltpu.VMEM((_RPW, _YD), jnp.float32),
            pltpu.VMEM((16, 1024), jnp.float32),
            pltpu.VMEM((_RPW, _YD), jnp.float32),
            pltpu.VMEM((_RPW, _YD), jnp.float32),
            pltpu.SemaphoreType.DMA,
        ],
    )(Y, y_pad, jnp.asarray(_IDX_TAB), jnp.asarray(_CC_FULL))

    X3 = X.reshape(_B, _SUB, 128)
    x_out = pl.pallas_call(
        _x_slab_body,
        grid=(_SUB // _CHUNK,),
        in_specs=[pl.BlockSpec((_B, _CHUNK, 128), lambda k: (0, k, 0))],
        out_specs=pl.BlockSpec((_B, _CHUNK, 128), lambda k: (0, k, 0)),
        out_shape=jax.ShapeDtypeStruct((_B, _SUB, 128), jnp.float32),
        compiler_params=pltpu.CompilerParams(
            dimension_semantics=("parallel",),
        ),
    )(X3)

    return (x_out.reshape(X.shape), y_out)


# speedup vs baseline: 1.0735x; 1.0094x over previous
import numpy as np
import jax
import jax.numpy as jnp
from jax import lax
from jax.experimental import pallas as pl
from jax.experimental.pallas import tpu as pltpu
from jax.experimental.pallas import tpu_sc as plsc

_B = 128
_ROW = 3 * 224 * 224
_SUB = _ROW // 128


_COEFF_BITS = [
    0x3d8aa995, 0x3b0adbe7, 0x32a77334, 0x3ea2194a, 0x3ab17549, 0x3e4e2364, 0x3f702c15, 0x36a32374,
    0x3e1112ae, 0x3f536550, 0x3dad7116, 0x3f610a35, 0x359e53f7, 0x3f6da517, 0x3f79a195, 0x3f7d02d6,
    0x3953ac92, 0x3f37666c, 0x3d8122ac, 0x3a7c868c, 0x3a8c3175, 0x3e66a9a8, 0x3f743a3c, 0x3f7fbb4d,
    0x3f769b79, 0x3f7fadd5, 0x3f7ca80e, 0x3f7d50da, 0x3e9bf821, 0x3f7e24ad, 0x385c80f2, 0x3f22b615,
    0x3e5c4d0d, 0x3f7f857d, 0x33d443b5, 0x3eee7eb0, 0x3f7fffe3, 0x3d26cebd, 0x38c67df6, 0x3f800000,
    0x3f7ffc2b, 0x33023ed7, 0x3f707e58, 0x3bbea683, 0x3ba47304, 0x3f63b612, 0x3eba6d66, 0x3bdfc38c,
    0x3f38aa7a, 0x3c0c7b62, 0x3dca5593, 0x3ec0ff48, 0x3f7fe5c3, 0x3e06f165, 0x3f796886, 0x3f49b8bd,
    0x3c23c8ab, 0x3b0c9b72, 0x2c72ff2c, 0x3f1a2af9, 0x3e2f970f, 0x3dde831c, 0x3f338f86, 0x3c43619c,
    0x3e1bc035, 0x3e9d6340, 0x3dd131a1, 0x39ff8fbf, 0x3d0db273, 0x3befa030, 0x3929564a, 0x3d63e6a1,
    0x3b681477, 0x3f7b5d60, 0x3dc9b188, 0x3f5c5b75, 0x3f7d7200, 0x3f503da5, 0x3ea70a9b, 0x3c3028b7,
    0x3da1c41c, 0x3aaf8c13, 0x3668c158, 0x2edad15c, 0x3f7efe4f, 0x3f67742e, 0x3f7e7a0e, 0x3cee4e69,
    0x3f6930dc, 0x3dab8b25, 0x3e188c64, 0x3a49b09e, 0x3f7d6765, 0x3d83424c, 0x3f150e74, 0x32d9477f,
    0x3f7be310, 0x3f79ec19, 0x3f7ffff9, 0x3f7fc3ab, 0x3f759076, 0x39735b79, 0x38aebadf, 0x3d699950,
    0x3e9f28b5, 0x3c7f65ec, 0x3f6a68f7, 0x3f7f1e62, 0x3f478cd4, 0x3f5cb538, 0x378da790, 0x3587b406,
    0x3d9e3e03, 0x3b1212b5, 0x3f7327ff, 0x3e22a57b, 0x3b359439, 0x3f78820e, 0x3f674a2a, 0x3e940a6a,
    0x3d7dac54, 0x3746b599, 0x3f7dae59, 0x3d837b1d, 0x3f7da727, 0x3c67fac8, 0x3e8294fd, 0x3ec57bc7,
]
_COEFFS = np.array(_COEFF_BITS, dtype=np.uint32).view(np.float32)
_PERM = np.array([
    83, 2, 65, 73, 78, 32, 15, 10, 71, 48, 85, 25, 116, 109, 114, 115,
    77, 28, 106, 93, 92, 0, 82, 49, 69, 87, 89, 104, 75, 4, 90, 60,
    84, 42, 21, 112, 72, 11, 20, 74, 103, 57, 17, 12, 125, 19, 22, 67,
    97, 18, 16, 27, 5, 86, 99, 23, 39, 100, 111, 26, 122, 7, 102, 29,
    126, 117, 98, 70, 120, 54, 9, 88, 96, 41, 53, 81, 13, 124, 105, 80,
    36, 37, 34, 6, 95, 46, 108, 62, 3, 52, 14, 66, 1, 123, 76, 61,
    110, 40, 44, 8, 58, 47, 33, 38, 55, 31, 119, 101, 118, 68, 64, 91,
    51, 79, 63, 24, 56, 107, 43, 127, 30, 121, 59, 94, 45, 113, 35, 50,
], dtype=np.int32)


_CHUNK = 168


def _x_slab_body(x_ref, o_ref):
    for i in range(_B):
        c = float(_COEFFS[i])
        p = int(_PERM[i])
        o_ref[i] = c * x_ref[i] + (1.0 - c) * x_ref[p]


_YD = 1000
_NW = 32
_RPW = _B // _NW
_FL = _RPW * _YD

_IDX_TAB = np.tile(_PERM.reshape(_NW, _RPW)[:, :1], (1, 16)).astype(np.int32)
_IDX_TAB[:, :_RPW] = _PERM.reshape(_NW, _RPW)
_CC_FULL = np.repeat(_COEFFS.reshape(_B, 1), _YD, axis=1)

_YOFFS = [16 * j for j in range(_YD // 16)] + [_YD - 16]


def _y_sc_body(y_hbm, ypad_hbm, idx_hbm, cc_hbm, o_hbm,
               idx_v, a_v, b_v, cc_v, o_v, sem):
    w = lax.axis_index("s") * 2 + lax.axis_index("c")
    base = w * _RPW
    pltpu.sync_copy(idx_hbm.at[w], idx_v)
    pltpu.sync_copy(y_hbm.at[pl.ds(base, _RPW)], a_v)
    pltpu.sync_copy(cc_hbm.at[pl.ds(base, _RPW)], cc_v)
    pltpu.async_copy(ypad_hbm.at[idx_v], b_v, sem).wait()
    for r in range(_RPW):
        for off in _YOFFS:
            sl = pl.ds(off, 16)
            cc = cc_v[r, sl]
            o_v[r, sl] = cc * a_v[r, sl] + (1.0 - cc) * b_v[r, sl]
    pltpu.sync_copy(o_v, o_hbm.at[pl.ds(base, _RPW)])


def kernel(X, Y):
    y_pad = jnp.pad(Y, ((0, 0), (0, 1024 - _YD)))
    y_out = pl.kernel(
        _y_sc_body,
        mesh=plsc.VectorSubcoreMesh(core_axis_name="c", subcore_axis_name="s"),
        out_type=jax.ShapeDtypeStruct((_B, _YD), jnp.float32),
        cost_estimate=pl.CostEstimate(
            flops=3 * _B * _YD, transcendentals=0,
            bytes_accessed=4 * 4 * _B * _YD),
        scratch_types=[
            pltpu.VMEM((16,), jnp.int32),
            pltpu.VMEM((_RPW, _YD), jnp.float32),
            pltpu.VMEM((16, 1024), jnp.float32),
            pltpu.VMEM((_RPW, _YD), jnp.float32),
            pltpu.VMEM((_RPW, _YD), jnp.float32),
            pltpu.SemaphoreType.DMA,
        ],
    )(Y, y_pad, jnp.asarray(_IDX_TAB), jnp.asarray(_CC_FULL))

    X3 = X.reshape(_B, _SUB, 128)
    x_out = pl.pallas_call(
        _x_slab_body,
        grid=(_SUB // _CHUNK,),
        in_specs=[pl.BlockSpec((_B, _CHUNK, 128), lambda k: (0, k, 0))],
        out_specs=pl.BlockSpec((_B, _CHUNK, 128), lambda k: (0, k, 0)),
        out_shape=jax.ShapeDtypeStruct((_B, _SUB, 128), jnp.float32),
        compiler_params=pltpu.CompilerParams(
            dimension_semantics=("parallel",),
        ),
    )(X3)

    return (x_out.reshape(X.shape), y_out)
